# baseline (speedup 1.0000x reference)
import functools

import jax
import jax.numpy as jnp
from jax import lax
from jax.experimental import pallas as pl
from jax.experimental.pallas import tpu as pltpu

T = 8


def _chart_kernel(y_ref, c_ref, cid_ref, *, K, M, BS):
    yp = y_ref[:, :M]
    c = c_ref[...]
    t1 = jnp.sum(yp * yp, axis=1, keepdims=True)
    dot = lax.dot_general(yp, c, (((1,), (1,)), ((), ())))
    t3 = jnp.sum(c * c, axis=1)[None, :]
    d2 = (t1 - 2.0 * dot) + t3
    minv = jnp.min(d2, axis=1, keepdims=True)
    iota = lax.broadcasted_iota(jnp.int32, d2.shape, 1)
    cid = jnp.min(jnp.where(d2 == minv, iota, K), axis=1)
    cid_ref[...] = cid.reshape(cid_ref.shape)


def _chart_assign(y, centroids):
    B, D = y.shape
    K, M = centroids.shape
    BS = min(512, B)
    NB = B // BS
    out = pl.pallas_call(
        functools.partial(_chart_kernel, K=K, M=M, BS=BS),
        grid=(NB,),
        in_specs=[
            pl.BlockSpec((BS, D), lambda i: (i, 0)),
            pl.BlockSpec((K, M), lambda i: (0, 0)),
        ],
        out_specs=pl.BlockSpec((1, 1, BS), lambda i: (i, 0, 0)),
        out_shape=jax.ShapeDtypeStruct((NB, 1, BS), jnp.int32),
    )(y, centroids)
    return out.reshape(B)


def _stage1_kernel(cid_ref, *refs, K, M, D, P):
    wps = refs[0:T]
    wph = refs[T:2 * T]
    ips = refs[2 * T:3 * T]
    bps = refs[3 * T:4 * T]
    ids = refs[4 * T:5 * T]
    bpsi = refs[5 * T:6 * T]
    bphi = refs[6 * T:7 * T]
    y_ref = refs[7 * T]
    z_ref, yb_ref, sw_ref, nc_ref = refs[7 * T + 1:]

    row_iota = lax.broadcasted_iota(jnp.int32, (1, 128), 1)
    col_iota = lax.broadcasted_iota(jnp.int32, (P, 1), 0)
    sw_vec = jnp.zeros((1, 128), jnp.int32)
    nc_vec = jnp.zeros((1, 128), jnp.int32)
    zs = []
    ybs = []
    for j in range(T):
        yj = y_ref[j:j + 1, :]
        Wj = wps[j][0]
        zj = lax.dot_general(yj, Wj, (((1,), (1,)), ((), ()))) + bpsi[j][0]
        zm = zj[:, :M]
        ipj = ips[j][0][:, :M]
        bpj = bps[j][0][:, :M]
        ddi = ipj - zm
        di = jnp.sum(ddi * ddi, axis=1, keepdims=True)
        ddb = bpj - zm
        db = jnp.sum(ddb * ddb, axis=1, keepdims=True)
        di_min = jnp.min(di)
        db_min = jnp.min(db)
        sw_j = db_min < di_min
        idx = jnp.min(jnp.where(db == db_min, col_iota, P))
        ids_j = ids[j][0]
        nc_j = jnp.sum(jnp.where(row_iota[:, :P] == idx, ids_j, 0))
        ybj = lax.dot_general(zj, wph[j][0], (((1,), (1,)), ((), ()))) + bphi[j][0]
        zs.append(zj)
        ybs.append(ybj)
        sw_vec = sw_vec + jnp.where(row_iota == j, sw_j.astype(jnp.int32), 0)
        nc_vec = nc_vec + jnp.where(row_iota == j, nc_j, 0)
    z_ref[...] = jnp.concatenate(zs, axis=0)
    yb_ref[...] = jnp.concatenate(ybs, axis=0)
    sw_ref[...] = sw_vec.reshape(1, 1, 128)
    nc_ref[...] = nc_vec.reshape(1, 1, 128)


def _stage1(cid, y, W_psi, W_phi, interior_points, boundary_points, ids3, bpsi3, bphi3):
    B, D = y.shape
    K, P, _ = interior_points.shape
    M = D // 2
    NB = B // T

    def gather_map(i, c, j):
        return (c[i * T + j], 0, 0)

    in_specs = []
    args = []
    for arr, spec3 in (
        (W_psi, (1, D, D)),
        (W_phi, (1, D, D)),
        (interior_points, (1, P, D)),
        (boundary_points, (1, P, D)),
        (ids3, (1, 1, P)),
        (bpsi3, (1, 1, D)),
        (bphi3, (1, 1, D)),
    ):
        for j in range(T):
            in_specs.append(pl.BlockSpec(spec3, functools.partial(gather_map, j=j)))
            args.append(arr)
    in_specs.append(pl.BlockSpec((T, D), lambda i, c: (i, 0)))
    args.append(y)

    out_specs = [
        pl.BlockSpec((T, D), lambda i, c: (i, 0)),
        pl.BlockSpec((T, D), lambda i, c: (i, 0)),
        pl.BlockSpec((1, 1, 128), lambda i, c: (i, 0, 0)),
        pl.BlockSpec((1, 1, 128), lambda i, c: (i, 0, 0)),
    ]
    out_shapes = [
        jax.ShapeDtypeStruct((B, D), jnp.float32),
        jax.ShapeDtypeStruct((B, D), jnp.float32),
        jax.ShapeDtypeStruct((NB, 1, 128), jnp.int32),
        jax.ShapeDtypeStruct((NB, 1, 128), jnp.int32),
    ]
    z, yb, sw_rows, nc_rows = pl.pallas_call(
        functools.partial(_stage1_kernel, K=K, M=M, D=D, P=P),
        grid_spec=pltpu.PrefetchScalarGridSpec(
            num_scalar_prefetch=1,
            grid=(NB,),
            in_specs=in_specs,
            out_specs=out_specs,
        ),
        out_shape=out_shapes,
    )(cid, *args)
    sw = sw_rows.reshape(NB, 128)[:, :T].reshape(B)
    nc = nc_rows.reshape(NB, 128)[:, :T].reshape(B)
    return z, yb, sw, nc


def _stage2_kernel(cid_ref, nc_ref, sw_ref, *refs, D):
    wps = refs[0:T]
    bpsi = refs[T:2 * T]
    yb_ref = refs[2 * T]
    z_ref = refs[2 * T + 1]
    fz_ref, fc_ref = refs[2 * T + 2:]

    i = pl.program_id(0)
    row_iota = lax.broadcasted_iota(jnp.int32, (1, 128), 1)
    fc_vec = jnp.zeros((1, 128), jnp.int32)
    rows = []
    for j in range(T):
        c_j = cid_ref[i * T + j]
        n_j = nc_ref[i * T + j]
        s_j = sw_ref[i * T + j]
        ybj = yb_ref[j:j + 1, :]
        zsj = lax.dot_general(ybj, wps[j][0], (((1,), (1,)), ((), ()))) + bpsi[j][0]
        zj = z_ref[j:j + 1, :]
        rows.append(jnp.where(s_j > 0, zsj, zj))
        fc_vec = fc_vec + jnp.where(row_iota == j, jnp.where(s_j > 0, n_j, c_j), 0)
    fz_ref[...] = jnp.concatenate(rows, axis=0)
    fc_ref[...] = fc_vec.reshape(1, 1, 128)


def _stage2(cid, nc, sw, yb, z, W_psi, bpsi3):
    B, D = z.shape
    NB = B // T

    def gather_map(i, c, n, s, j):
        return (n[i * T + j], 0, 0)

    in_specs = []
    args = []
    for j in range(T):
        in_specs.append(pl.BlockSpec((1, D, D), functools.partial(gather_map, j=j)))
        args.append(W_psi)
    for j in range(T):
        in_specs.append(pl.BlockSpec((1, 1, D), functools.partial(gather_map, j=j)))
        args.append(bpsi3)
    in_specs.append(pl.BlockSpec((T, D), lambda i, c, n, s: (i, 0)))
    args.append(yb)
    in_specs.append(pl.BlockSpec((T, D), lambda i, c, n, s: (i, 0)))
    args.append(z)

    out_specs = [
        pl.BlockSpec((T, D), lambda i, c, n, s: (i, 0)),
        pl.BlockSpec((1, 1, 128), lambda i, c, n, s: (i, 0, 0)),
    ]
    out_shapes = [
        jax.ShapeDtypeStruct((B, D), jnp.float32),
        jax.ShapeDtypeStruct((NB, 1, 128), jnp.int32),
    ]
    fz, fc_rows = pl.pallas_call(
        functools.partial(_stage2_kernel, D=D),
        grid_spec=pltpu.PrefetchScalarGridSpec(
            num_scalar_prefetch=3,
            grid=(NB,),
            in_specs=in_specs,
            out_specs=out_specs,
        ),
        out_shape=out_shapes,
    )(cid, nc, sw, *args)
    fc = fc_rows.reshape(NB, 128)[:, :T].reshape(B)
    return fc, fz


def kernel(y, centroids, interior_points, boundary_points, boundary_new_chart_ids,
           W_psi, b_psi, W_phi, b_phi):
    B, D = y.shape
    K, P = boundary_new_chart_ids.shape
    ids3 = boundary_new_chart_ids.reshape(K, 1, P)
    bpsi3 = b_psi.reshape(K, 1, D)
    bphi3 = b_phi.reshape(K, 1, D)

    cid = _chart_assign(y, centroids)
    z, yb, sw, nc = _stage1(cid, y, W_psi, W_phi, interior_points, boundary_points,
                            ids3, bpsi3, bphi3)
    fc, fz = _stage2(cid, nc, sw, yb, z, W_psi, bpsi3)
    return fc, fz


# speedup vs baseline: 1.6790x; 1.6790x over previous
import functools

import jax
import jax.numpy as jnp
from jax import lax
from jax.experimental import pallas as pl
from jax.experimental.pallas import tpu as pltpu

T = 16


def _chart_kernel(y_ref, c_ref, cid_ref, *, K, M):
    yp = y_ref[:, :M]
    c = c_ref[...]
    t1 = jnp.sum(yp * yp, axis=1, keepdims=True)
    dot = lax.dot_general(yp, c, (((1,), (1,)), ((), ())))
    t3 = jnp.sum(c * c, axis=1)[None, :]
    d2 = (t1 - 2.0 * dot) + t3
    minv = jnp.min(d2, axis=1, keepdims=True)
    iota = lax.broadcasted_iota(jnp.int32, d2.shape, 1)
    cid = jnp.min(jnp.where(d2 == minv, iota, K), axis=1)
    cid_ref[...] = cid.reshape(cid_ref.shape)


def _chart_assign(y, centroids):
    B, D = y.shape
    K, M = centroids.shape
    BS = min(512, B)
    NB = B // BS
    out = pl.pallas_call(
        functools.partial(_chart_kernel, K=K, M=M),
        grid=(NB,),
        in_specs=[
            pl.BlockSpec((BS, D), lambda i: (i, 0)),
            pl.BlockSpec((K, M), lambda i: (0, 0)),
        ],
        out_specs=pl.BlockSpec((1, 1, BS), lambda i: (i, 0, 0)),
        out_shape=jax.ShapeDtypeStruct((NB, 1, BS), jnp.int32),
    )(y, centroids)
    return out.reshape(B)


def _stage1_kernel(cid_ref, *refs, K, M, D, P):
    wpk = refs[0:T]
    ips = refs[T:2 * T]
    bps = refs[2 * T:3 * T]
    prm = refs[3 * T:4 * T]
    y_ref = refs[4 * T]
    z_ref, yb_ref, sw_ref, nc_ref = refs[4 * T + 1:]

    y_blk = y_ref[...]
    zs = []
    for j in range(T):
        Wj = wpk[j][0, :D, :]
        bj = lax.bitcast_convert_type(prm[j][0, :, :D], jnp.float32)
        zs.append(lax.dot_general(y_blk[j:j + 1, :], Wj,
                                  (((1,), (1,)), ((), ()))) + bj)
    z = jnp.concatenate(zs, axis=0)

    ip_all = jnp.concatenate([r[0][:, :M] for r in ips], axis=0)
    bp_all = jnp.concatenate([r[0][:, :M] for r in bps], axis=0)
    zm = z[:, :M]
    zm_exp = jnp.broadcast_to(zm[:, None, :], (T, P, M)).reshape(T * P, M)
    ddi = ip_all - zm_exp
    di = jnp.sum(ddi * ddi, axis=1, keepdims=True)
    ddb = bp_all - zm_exp
    db = jnp.sum(ddb * ddb, axis=1, keepdims=True)
    di2 = di.reshape(T, P)
    db2 = db.reshape(T, P)
    di_min = jnp.min(di2, axis=1, keepdims=True)
    db_min = jnp.min(db2, axis=1, keepdims=True)
    sw = (db_min < di_min)
    piota = lax.broadcasted_iota(jnp.int32, (T, P), 1)
    idx = jnp.min(jnp.where(db2 == db_min, piota, P), axis=1, keepdims=True)

    ids_all = jnp.concatenate([r[0][:, 2 * D:2 * D + P] for r in prm], axis=0)
    nc = jnp.sum(jnp.where(piota == idx, ids_all, 0), axis=1, keepdims=True)

    ybs = []
    for j in range(T):
        Vj = wpk[j][0, D:2 * D, :]
        cj = lax.bitcast_convert_type(prm[j][0, :, D:2 * D], jnp.float32)
        ybs.append(lax.dot_general(z[j:j + 1, :], Vj,
                                   (((1,), (1,)), ((), ()))) + cj)
    yb = jnp.concatenate(ybs, axis=0)

    z_ref[...] = z
    yb_ref[...] = yb
    lane = lax.broadcasted_iota(jnp.int32, (T, 128), 1)
    samp = lax.broadcasted_iota(jnp.int32, (T, 128), 0)
    sw_row = jnp.sum(jnp.where(lane == samp, sw.astype(jnp.int32), 0), axis=0, keepdims=True)
    nc_row = jnp.sum(jnp.where(lane == samp, nc, 0), axis=0, keepdims=True)
    sw_ref[...] = sw_row.reshape(1, 1, 128)
    nc_ref[...] = nc_row.reshape(1, 1, 128)


def _stage1(cid, y, W_pack, interior_points, boundary_points, params):
    B, D = y.shape
    K, P, _ = interior_points.shape
    M = D // 2
    NB = B // T

    def gather_map(i, c, j):
        return (c[i * T + j], 0, 0)

    in_specs = []
    args = []
    for arr, spec3 in (
        (W_pack, (1, 2 * D, D)),
        (interior_points, (1, P, D)),
        (boundary_points, (1, P, D)),
        (params, (1, 1, 4 * D)),
    ):
        for j in range(T):
            in_specs.append(pl.BlockSpec(spec3, functools.partial(gather_map, j=j)))
            args.append(arr)
    in_specs.append(pl.BlockSpec((T, D), lambda i, c: (i, 0)))
    args.append(y)

    out_specs = [
        pl.BlockSpec((T, D), lambda i, c: (i, 0)),
        pl.BlockSpec((T, D), lambda i, c: (i, 0)),
        pl.BlockSpec((1, 1, 128), lambda i, c: (i, 0, 0)),
        pl.BlockSpec((1, 1, 128), lambda i, c: (i, 0, 0)),
    ]
    out_shapes = [
        jax.ShapeDtypeStruct((B, D), jnp.float32),
        jax.ShapeDtypeStruct((B, D), jnp.float32),
        jax.ShapeDtypeStruct((NB, 1, 128), jnp.int32),
        jax.ShapeDtypeStruct((NB, 1, 128), jnp.int32),
    ]
    z, yb, sw_rows, nc_rows = pl.pallas_call(
        functools.partial(_stage1_kernel, K=K, M=M, D=D, P=P),
        grid_spec=pltpu.PrefetchScalarGridSpec(
            num_scalar_prefetch=1,
            grid=(NB,),
            in_specs=in_specs,
            out_specs=out_specs,
        ),
        out_shape=out_shapes,
    )(cid, *args)
    sw = sw_rows.reshape(NB, 128)[:, :T].reshape(B)
    nc = nc_rows.reshape(NB, 128)[:, :T].reshape(B)
    return z, yb, sw, nc


def _stage2_kernel(cid_ref, nc_ref, sw_ref, *refs, D):
    wps = refs[0:T]
    prm = refs[T:2 * T]
    yb_ref = refs[2 * T]
    z_ref = refs[2 * T + 1]
    fz_ref, fc_ref = refs[2 * T + 2:]

    i = pl.program_id(0)
    rows = []
    fcs = []
    for j in range(T):
        c_j = cid_ref[i * T + j]
        n_j = nc_ref[i * T + j]
        s_j = sw_ref[i * T + j]
        bj = lax.bitcast_convert_type(prm[j][0, :, :D], jnp.float32)
        zsj = lax.dot_general(yb_ref[j:j + 1, :], wps[j][0],
                              (((1,), (1,)), ((), ()))) + bj
        rows.append(jnp.where(s_j > 0, zsj, z_ref[j:j + 1, :]))
        fcs.append(jnp.where(s_j > 0, n_j, c_j))
    fz_ref[...] = jnp.concatenate(rows, axis=0)
    lane = lax.broadcasted_iota(jnp.int32, (1, 128), 1)
    fc_vec = jnp.zeros((1, 128), jnp.int32)
    for j in range(T):
        fc_vec = fc_vec + jnp.where(lane == j, fcs[j], 0)
    fc_ref[...] = fc_vec.reshape(1, 1, 128)


def _stage2(cid, nc, sw, yb, z, W_psi, params):
    B, D = z.shape
    NB = B // T

    def gather_map(i, c, n, s, j):
        return (n[i * T + j], 0, 0)

    in_specs = []
    args = []
    for j in range(T):
        in_specs.append(pl.BlockSpec((1, D, D), functools.partial(gather_map, j=j)))
        args.append(W_psi)
    for j in range(T):
        in_specs.append(pl.BlockSpec((1, 1, 4 * D), functools.partial(gather_map, j=j)))
        args.append(params)
    in_specs.append(pl.BlockSpec((T, D), lambda i, c, n, s: (i, 0)))
    args.append(yb)
    in_specs.append(pl.BlockSpec((T, D), lambda i, c, n, s: (i, 0)))
    args.append(z)

    out_specs = [
        pl.BlockSpec((T, D), lambda i, c, n, s: (i, 0)),
        pl.BlockSpec((1, 1, 128), lambda i, c, n, s: (i, 0, 0)),
    ]
    out_shapes = [
        jax.ShapeDtypeStruct((B, D), jnp.float32),
        jax.ShapeDtypeStruct((NB, 1, 128), jnp.int32),
    ]
    fz, fc_rows = pl.pallas_call(
        functools.partial(_stage2_kernel, D=D),
        grid_spec=pltpu.PrefetchScalarGridSpec(
            num_scalar_prefetch=3,
            grid=(NB,),
            in_specs=in_specs,
            out_specs=out_specs,
        ),
        out_shape=out_shapes,
    )(cid, nc, sw, *args)
    fc = fc_rows.reshape(NB, 128)[:, :T].reshape(B)
    return fc, fz


def kernel(y, centroids, interior_points, boundary_points, boundary_new_chart_ids,
           W_psi, b_psi, W_phi, b_phi):
    B, D = y.shape
    K, P = boundary_new_chart_ids.shape

    W_pack = jnp.concatenate([W_psi, W_phi], axis=1)
    pad = jnp.zeros((K, 4 * D - 2 * D - P), jnp.int32)
    params = jnp.concatenate([
        lax.bitcast_convert_type(b_psi, jnp.int32),
        lax.bitcast_convert_type(b_phi, jnp.int32),
        boundary_new_chart_ids,
        pad,
    ], axis=1).reshape(K, 1, 4 * D)

    cid = _chart_assign(y, centroids)
    z, yb, sw, nc = _stage1(cid, y, W_pack, interior_points, boundary_points, params)
    fc, fz = _stage2(cid, nc, sw, yb, z, W_psi, params)
    return fc, fz
